# baseline (speedup 1.0000x reference)
import jax
import jax.numpy as jnp
from jax.experimental import pallas as pl
from jax.experimental.pallas import tpu as pltpu

_NV = 8
_EPS = 1e-5


def _compute(x_ro, ln_g, ln_b, W1, b1, W2, b2, gW, gb, out_ref):
    wsum = None
    corr = None
    gates = []
    for v in range(_NV):
        xv = x_ro[v]
        mu_v = jnp.mean(xv, axis=0, keepdims=True)
        msq_v = jnp.mean(xv * xv, axis=0, keepdims=True)
        r_v = jax.lax.rsqrt(msq_v - mu_v * mu_v + _EPS)
        term = xv * r_v
        cterm = mu_v * r_v
        wsum = term if wsum is None else wsum + term
        corr = cterm if corr is None else corr + cterm
        pooled_v = jnp.mean(xv, axis=1, keepdims=True)
        logit_v = jax.lax.dot_general(
            gW, pooled_v, (((1,), (0,)), ((), ())),
            preferred_element_type=jnp.float32) + gb
        gates.append(jax.nn.sigmoid(logit_v))

    s = ln_g * ((wsum - corr) * (1.0 / _NV)) + ln_b

    h1 = jax.lax.dot_general(
        W1, s, (((0,), (0,)), ((), ())),
        preferred_element_type=jnp.float32) + b1

    mu2 = jnp.mean(h1, axis=0, keepdims=True)
    var2 = jnp.mean((h1 - mu2) ** 2, axis=0, keepdims=True)
    a = jnp.maximum((h1 - mu2) * jax.lax.rsqrt(var2 + _EPS), 0.0)

    h2 = jax.lax.dot_general(
        W2, a, (((0,), (0,)), ((), ())),
        preferred_element_type=jnp.float32) + b2

    for v in range(_NV):
        out_ref[v] = x_ro[v] + gates[v] * h2


def _body(x_hbm, ln_g_ref, ln_b_ref, W1_ref, b1_ref, W2_ref, b2_ref,
          gW_ref, gb_ref, out_hbm,
          xbuf0, xbuf1, obuf0, obuf1, si0, si1, so0, so1):
    j = pl.program_id(0)
    nsteps = pl.num_programs(0)
    blk0 = 2 * j
    blk1 = 2 * j + 1

    def in_copy(blk, buf, sem):
        return pltpu.make_async_copy(
            x_hbm.at[pl.ds(blk * _NV, _NV)], buf, sem)

    def out_copy(blk, buf, sem):
        return pltpu.make_async_copy(
            buf, out_hbm.at[pl.ds(blk * _NV, _NV)], sem)

    args = (ln_g_ref[...], ln_b_ref[...], W1_ref[...], b1_ref[...],
            W2_ref[...], b2_ref[...], gW_ref[...], gb_ref[...])

    @pl.when(j == 0)
    def _():
        in_copy(0, xbuf0, si0).start()
        in_copy(1, xbuf1, si1).start()

    in_copy(blk0, xbuf0, si0).wait()
    @pl.when(j >= 1)
    def _():
        out_copy(blk0 - 2, obuf0, so0).wait()

    _compute(xbuf0, *args, obuf0)
    out_copy(blk0, obuf0, so0).start()

    @pl.when(j + 1 < nsteps)
    def _():
        in_copy(blk0 + 2, xbuf0, si0).start()

    in_copy(blk1, xbuf1, si1).wait()
    @pl.when(j >= 1)
    def _():
        out_copy(blk1 - 2, obuf1, so1).wait()

    _compute(xbuf1, *args, obuf1)
    out_copy(blk1, obuf1, so1).start()

    @pl.when(j + 1 < nsteps)
    def _():
        in_copy(blk1 + 2, xbuf1, si1).start()

    @pl.when(j == nsteps - 1)
    def _():
        out_copy(blk0, obuf0, so0).wait()
        out_copy(blk1, obuf1, so1).wait()


@jax.jit
def kernel(x, data_key, ln_g, ln_b, W1, b1, W2, b2, gate_W, gate_b):
    B, C, T = x.shape
    n_steps = B // _NV // 2

    in_specs = [
        pl.BlockSpec(memory_space=pl.ANY),
        pl.BlockSpec((C, 1), lambda i: (0, 0)),
        pl.BlockSpec((C, 1), lambda i: (0, 0)),
        pl.BlockSpec((C, C), lambda i: (0, 0)),
        pl.BlockSpec((C, 1), lambda i: (0, 0)),
        pl.BlockSpec((C, C), lambda i: (0, 0)),
        pl.BlockSpec((C, 1), lambda i: (0, 0)),
        pl.BlockSpec((C, C), lambda i: (0, 0)),
        pl.BlockSpec((C, 1), lambda i: (0, 0)),
    ]

    return pl.pallas_call(
        _body,
        grid=(n_steps,),
        in_specs=in_specs,
        out_specs=pl.BlockSpec(memory_space=pl.ANY),
        out_shape=jax.ShapeDtypeStruct((B, C, T), x.dtype),
        scratch_shapes=[
            pltpu.VMEM((_NV, C, T), jnp.float32),
            pltpu.VMEM((_NV, C, T), jnp.float32),
            pltpu.VMEM((_NV, C, T), jnp.float32),
            pltpu.VMEM((_NV, C, T), jnp.float32),
            pltpu.SemaphoreType.DMA,
            pltpu.SemaphoreType.DMA,
            pltpu.SemaphoreType.DMA,
            pltpu.SemaphoreType.DMA,
        ],
        compiler_params=pltpu.CompilerParams(
            dimension_semantics=("arbitrary",),
        ),
    )(x, ln_g.reshape(C, 1), ln_b.reshape(C, 1), W1, b1.reshape(C, 1),
      W2, b2.reshape(C, 1), gate_W, gate_b.reshape(C, 1))
